# baseline (speedup 1.0000x reference)
import functools

import jax
import jax.numpy as jnp
from jax import lax
from jax.experimental import pallas as pl
from jax.experimental.pallas import tpu as pltpu
from jax.experimental.pallas import tpu_sc as plsc

P, C, D = 64, 64, 256
S, K = 2048, 2
NC, NS, L = 2, 16, 16
TPW = S // NS
DH = D // NC
PPW = P // NS
E = K * TPW

_DIMNUMS = lax.GatherDimensionNumbers(
    offset_dims=(), collapsed_slice_dims=(0,), start_index_map=(0,))


def _permute(v, idx):
    return lax.gather(v, idx[:, None], _DIMNUMS, (1,),
                      mode=lax.GatherScatterMode.PROMISE_IN_BOUNDS)


_sc_mesh = plsc.VectorSubcoreMesh(core_axis_name="c", subcore_axis_name="s")


@functools.partial(
    pl.kernel,
    mesh=_sc_mesh,
    out_type=jax.ShapeDtypeStruct((S, D), jnp.float32),
    scratch_types=[
        pltpu.VMEM((E, DH), jnp.float32),
        pltpu.VMEM((TPW, DH), jnp.float32),
        pltpu.VMEM((E, C), jnp.float32),
        pltpu.VMEM((E,), jnp.int32),
        pltpu.VMEM((PPW, DH), jnp.float32),
        pltpu.VMEM_SHARED((P, DH), jnp.float32),
    ],
)
def _sc_all(idx_hbm, keys_hbm, values_hbm, q_hbm, states_hbm, out_hbm,
            sbuf, vbuf, kbuf, ibuf, tbuf, shared):
    cid = lax.axis_index("c")
    sid = lax.axis_index("s")
    t0 = sid * TPW
    c0 = cid * DH
    p0 = sid * PPW
    e0 = t0 * K

    pltpu.sync_copy(states_hbm.at[pl.ds(p0 * C, PPW * C), pl.ds(c0, DH)], sbuf)
    for i in range(PPW):
        def srow(c, acc):
            return tuple(acc[j] + sbuf[i * C + c, pl.ds(j * L, L)]
                         for j in range(DH // L))
        acc0 = tuple(jnp.zeros((L,), jnp.float32) for _ in range(DH // L))
        acc = lax.fori_loop(0, C, srow, acc0)
        for j in range(DH // L):
            tbuf[i, pl.ds(j * L, L)] = acc[j] * (1.0 / C)
    pltpu.sync_copy(tbuf, shared.at[pl.ds(p0, PPW)])
    pltpu.sync_copy(idx_hbm.at[pl.ds(e0, E)], ibuf)
    pltpu.sync_copy(keys_hbm.at[pl.ds(e0, E)], kbuf)
    pltpu.sync_copy(values_hbm.at[pl.ds(t0, TPW), pl.ds(c0, DH)], vbuf)
    plsc.subcore_barrier()

    iota = lax.iota(jnp.int32, L)
    bidx = tuple(iota ^ sh for sh in (8, 4, 2, 1))
    UT = 4

    def grp(g, carry):
        tb = g * UT
        for u in range(UT):
            t = tb + u
            for k in range(K):
                e = t * K + k
                a = (kbuf[e, pl.ds(0, L)] + kbuf[e, pl.ds(L, L)]
                     + kbuf[e, pl.ds(2 * L, L)] + kbuf[e, pl.ds(3 * L, L)])
                for bv in bidx:
                    a = a + _permute(a, bv)
                sv = a * (1.0 / C)
                for j in range(DH // L):
                    sl = pl.ds(j * L, L)
                    sbuf[e, sl] = sv * vbuf[t, sl]
        return carry

    lax.fori_loop(0, TPW // UT, grp, 0)
    pltpu.sync_copy(sbuf, shared.at[ibuf], add=True)
    pltpu.sync_copy(q_hbm.at[pl.ds(t0, TPW), pl.ds(c0, DH)], vbuf)
    plsc.subcore_barrier()

    pltpu.sync_copy(shared.at[ibuf], sbuf)

    def tok(t, carry):
        for j in range(DH // L):
            sl = pl.ds(j * L, L)
            g = sbuf[t * K, sl] + sbuf[t * K + 1, sl]
            vbuf[t, sl] = g * vbuf[t, sl]
        return carry

    lax.fori_loop(0, TPW, tok, 0)
    pltpu.sync_copy(vbuf, out_hbm.at[pl.ds(t0, TPW), pl.ds(c0, DH)])


def kernel(partition_indices, keys, values, queries, states):
    b, s, k = partition_indices.shape
    assert (b, s, k) == (1, S, K)
    idx2 = partition_indices.reshape(S * K).astype(jnp.int32)
    keys2 = keys.reshape(S * K, C)
    values2 = values.reshape(S, D)
    queries2 = queries.reshape(S, D)
    states2 = states.reshape(P * C, D)

    out2 = _sc_all(idx2, keys2, values2, queries2, states2)
    return out2.reshape(1, S, D)
